# baseline (speedup 1.0000x reference)
import functools

import jax
import jax.numpy as jnp
from jax.experimental import pallas as pl

_B = 4096
_TR = 256
_NEG = float("-inf")


def _expand_offdiag(g):
    B = g.shape[0]
    z = jnp.concatenate(
        [jnp.full((B - 1, 1), _NEG, jnp.float32), g.reshape(B - 1, B)], axis=1
    ).reshape(-1)
    return jnp.concatenate([z, jnp.full((1,), _NEG, jnp.float32)]).reshape(B, B)


def _row_kernel(l_ref, g_ref, pos_ref, neg_ref):
    i = pl.program_id(0)
    L = l_ref[...]
    s = g_ref[...] + (L + 1e-4)
    c = jax.lax.broadcasted_iota(jnp.int32, L.shape, 1)
    r = jax.lax.broadcasted_iota(jnp.int32, L.shape, 0) + i * _TR
    m = jnp.max(s, axis=1, keepdims=True)
    idx = jnp.min(jnp.where(s == m, c, _B), axis=1, keepdims=True)
    neg_ref[...] = jnp.sum(jnp.where(c == idx, L, 0.0), axis=1, keepdims=True)
    pos_ref[...] = jnp.sum(jnp.where(c == r, L, 0.0), axis=1, keepdims=True)


def _col_kernel(l_ref, g_ref, neg_ref):
    j = pl.program_id(0)
    L = l_ref[...]
    s = g_ref[...] + (L + 1e-4)
    r = jax.lax.broadcasted_iota(jnp.int32, L.shape, 0)
    m = jnp.max(s, axis=0, keepdims=True)
    idx = jnp.min(jnp.where(s == m, r, _B), axis=0, keepdims=True)
    v = jnp.sum(jnp.where(r == idx, L, 0.0), axis=0, keepdims=True)
    neg_ref[...] = v.T


@functools.partial(jax.jit, static_argnames=())
def kernel(logits):
    B = _B
    g1 = _expand_offdiag(jax.random.gumbel(jax.random.key(1), (B, B - 1), jnp.float32))
    g2 = _expand_offdiag(jax.random.gumbel(jax.random.key(2), (B, B - 1), jnp.float32))
    g2t = g2.T

    pos, neg1 = pl.pallas_call(
        _row_kernel,
        grid=(B // _TR,),
        in_specs=[
            pl.BlockSpec((_TR, B), lambda i: (i, 0)),
            pl.BlockSpec((_TR, B), lambda i: (i, 0)),
        ],
        out_specs=[
            pl.BlockSpec((_TR, 1), lambda i: (i, 0)),
            pl.BlockSpec((_TR, 1), lambda i: (i, 0)),
        ],
        out_shape=[
            jax.ShapeDtypeStruct((B, 1), jnp.float32),
            jax.ShapeDtypeStruct((B, 1), jnp.float32),
        ],
    )(logits, g1)

    neg2 = pl.pallas_call(
        _col_kernel,
        grid=(B // _TR,),
        in_specs=[
            pl.BlockSpec((B, _TR), lambda j: (0, j)),
            pl.BlockSpec((B, _TR), lambda j: (0, j)),
        ],
        out_specs=pl.BlockSpec((_TR, 1), lambda j: (j, 0)),
        out_shape=jax.ShapeDtypeStruct((B, 1), jnp.float32),
    )(logits, g2t)

    data = jnp.concatenate([pos, neg1, pos, neg2], axis=0)
    ones = jnp.ones((B,), jnp.float32)
    zeros = jnp.zeros((B,), jnp.float32)
    label = jnp.concatenate([ones, zeros, ones, zeros], axis=0)
    return (data, label)


# speedup vs baseline: 2.9301x; 2.9301x over previous
import jax
import jax.numpy as jnp
from jax.experimental import pallas as pl

_B = 4096
_TR = 256
_NEG = float("-inf")
_TINY = float(jnp.finfo(jnp.float32).tiny)


def _gumbel_from_index(idx, seed):
    idx = idx.astype(jnp.uint32)
    ks0 = jnp.uint32(0)
    ks1 = jnp.uint32(seed)
    ks2 = ks0 ^ ks1 ^ jnp.uint32(0x1BD11BDA)

    def rounds(x0, x1, rots):
        for r in rots:
            x0 = x0 + x1
            x1 = (x1 << jnp.uint32(r)) | (x1 >> jnp.uint32(32 - r))
            x1 = x1 ^ x0
        return x0, x1

    ra = (13, 15, 26, 6)
    rb = (17, 29, 16, 24)
    x0 = jnp.zeros_like(idx) + ks0
    x1 = idx + ks1
    x0, x1 = rounds(x0, x1, ra)
    x0 = x0 + ks1
    x1 = x1 + ks2 + jnp.uint32(1)
    x0, x1 = rounds(x0, x1, rb)
    x0 = x0 + ks2
    x1 = x1 + ks0 + jnp.uint32(2)
    x0, x1 = rounds(x0, x1, ra)
    x0 = x0 + ks0
    x1 = x1 + ks1 + jnp.uint32(3)
    x0, x1 = rounds(x0, x1, rb)
    x0 = x0 + ks1
    x1 = x1 + ks2 + jnp.uint32(4)
    x0, x1 = rounds(x0, x1, ra)
    x0 = x0 + ks2
    x1 = x1 + ks0 + jnp.uint32(5)
    bits = x0 ^ x1

    float_bits = (bits >> jnp.uint32(9)) | jnp.uint32(0x3F800000)
    f = jax.lax.bitcast_convert_type(float_bits, jnp.float32) - jnp.float32(1.0)
    u = jnp.maximum(jnp.float32(_TINY), f + jnp.float32(_TINY))
    return -jnp.log(-jnp.log(u))


def _row_kernel(l_ref, pos_ref, neg_ref):
    i = pl.program_id(0)
    L = l_ref[...]
    c = jax.lax.broadcasted_iota(jnp.int32, L.shape, 1)
    r = jax.lax.broadcasted_iota(jnp.int32, L.shape, 0) + i * _TR
    n = r * (_B - 1) + c - (c > r).astype(jnp.int32)
    g = _gumbel_from_index(n, 1)
    s = jnp.where(c == r, _NEG, g + (L + 1e-4))
    m = jnp.max(s, axis=1, keepdims=True)
    idx = jnp.min(jnp.where(s == m, c, _B), axis=1, keepdims=True)
    neg_ref[...] = jnp.sum(jnp.where(c == idx, L, 0.0), axis=1, keepdims=True)
    pos_ref[...] = jnp.sum(jnp.where(c == r, L, 0.0), axis=1, keepdims=True)


def _col_kernel(l_ref, neg_ref):
    j = pl.program_id(0)
    L = l_ref[...]
    r = jax.lax.broadcasted_iota(jnp.int32, L.shape, 0)
    c = jax.lax.broadcasted_iota(jnp.int32, L.shape, 1) + j * _TR
    n = c * (_B - 1) + r - (r > c).astype(jnp.int32)
    g = _gumbel_from_index(n, 2)
    s = jnp.where(r == c, _NEG, g + (L + 1e-4))
    m = jnp.max(s, axis=0, keepdims=True)
    idx = jnp.min(jnp.where(s == m, r, _B), axis=0, keepdims=True)
    v = jnp.sum(jnp.where(r == idx, L, 0.0), axis=0, keepdims=True)
    neg_ref[...] = v.T


def kernel(logits):
    B = _B
    pos, neg1 = pl.pallas_call(
        _row_kernel,
        grid=(B // _TR,),
        in_specs=[pl.BlockSpec((_TR, B), lambda i: (i, 0))],
        out_specs=[
            pl.BlockSpec((_TR, 1), lambda i: (i, 0)),
            pl.BlockSpec((_TR, 1), lambda i: (i, 0)),
        ],
        out_shape=[
            jax.ShapeDtypeStruct((B, 1), jnp.float32),
            jax.ShapeDtypeStruct((B, 1), jnp.float32),
        ],
    )(logits)

    neg2 = pl.pallas_call(
        _col_kernel,
        grid=(B // _TR,),
        in_specs=[pl.BlockSpec((B, _TR), lambda j: (0, j))],
        out_specs=pl.BlockSpec((_TR, 1), lambda j: (j, 0)),
        out_shape=jax.ShapeDtypeStruct((B, 1), jnp.float32),
    )(logits)

    data = jnp.concatenate([pos, neg1, pos, neg2], axis=0)
    ones = jnp.ones((B,), jnp.float32)
    zeros = jnp.zeros((B,), jnp.float32)
    label = jnp.concatenate([ones, zeros, ones, zeros], axis=0)
    return (data, label)
